# baseline (speedup 1.0000x reference)
import functools

import jax
import jax.numpy as jnp
from jax import lax
from jax.experimental import pallas as pl
from jax.experimental.pallas import tpu as pltpu
from jax.experimental.pallas import tpu_sc as plsc

_TEMPERATURE = 0.5
_K = 64
_NC = 10
_BATCH = 16384
_DIM = 128
_ROWS = 2 * _K * _NC
_BLK = 256
_CH = 16
_NCHUNK = _BATCH // _CH
_CPW = _NCHUNK // 16
_RW = _NC * _CH


def _iota16():
    return lax.iota(jnp.int32, 16)


def _mono_key(v):
    u = lax.bitcast_convert_type(v, jnp.uint32)
    return jnp.where(v < 0.0, ~u, u | jnp.uint32(0x80000000))


def _count_slices(nsl, load, pred):
    cnt = jnp.zeros((16,), jnp.int32)
    for i in range(nsl):
        cnt = cnt + jnp.where(pred(load(i), i), 1, 0)
    return jnp.sum(cnt)


def _bisect_key(load, nsl, k, lower=None):
    def body(i, acc):
        bit = lax.shift_left(jnp.uint32(1), (31 - i).astype(jnp.uint32))
        test = acc | bit
        if lower is None:
            c = _count_slices(nsl, load, lambda v, _: v >= test)
        else:
            c = lax.cond(
                test <= lower,
                lambda: jnp.int32(k),
                lambda: _count_slices(nsl, load, lambda v, _: v >= test),
            )
        return jnp.where(c >= k, test, acc)

    return lax.fori_loop(0, 32, body, jnp.uint32(0))


def _bisect_idx(load_key, load_idx, nsl, v, kp, nbits):
    def body(i, acc):
        b = (nbits - 1 - i).astype(jnp.int32)
        low = lax.shift_left(jnp.int32(1), b) - 1
        test = acc | low
        c = _count_slices(
            nsl, load_key,
            lambda kv, j: (kv == v) & (load_idx(j) <= test),
        )
        return jnp.where(c >= kp, acc, acc | (low + 1))

    return lax.fori_loop(0, nbits, body, jnp.int32(0))


def _select64(key_ref, idx_vec_fn, n, nbits, out_ref, lower=None):
    nsl = n // 16

    def loadk(i):
        return plsc.bitcast(key_ref[pl.ds(i * 16, 16)], jnp.uint32)

    v = _bisect_key(loadk, nsl, 64, lower=lower)
    ngt = _count_slices(nsl, loadk, lambda kv, _: kv > v)
    kp = 64 - ngt
    g = _bisect_idx(loadk, idx_vec_fn, nsl, v, kp, nbits)

    def emit(i, run):
        kv = plsc.bitcast(key_ref[pl.ds(i * 16, 16)], jnp.uint32)
        idx = idx_vec_fn(i)
        m = (kv > v) | ((kv == v) & (idx <= g))
        incl = plsc.cumsum(m.astype(jnp.int32))
        slots = run + incl - 1
        plsc.store_scatter(out_ref, [slots], idx, mask=m)
        return run + plsc.all_reduce_population_count(m)

    lax.fori_loop(0, nsl, emit, jnp.zeros((16,), jnp.int32), unroll=2)
    return v


def _sc_body(probf_hbm, zi_hbm, zj_hbm, u_hbm,
             chunkflat, slab, cmbuf, keyb, bidxb, idchunks, zidx,
             zrows, zrows2, cm_shared, sem, sem2):
    s = lax.axis_index("s")
    it = _iota16()

    with jax.named_scope("ph0_load"):
        pltpu.sync_copy(probf_hbm.at[pl.ds(s * _CPW * _RW, _CPW * _RW)],
                        chunkflat.at[pl.ds(0, _CPW * _RW)])

    def p0(q, _):
        m = chunkflat[pl.ds(q * _RW, 16)]
        for p in range(1, _CH):
            m = jnp.maximum(m, chunkflat[pl.ds(q * _RW + p * _NC, 16)])
        key = plsc.bitcast(_mono_key(m), jnp.int32)
        plsc.store_scatter(slab, [it * _CPW + q], key)
        return 0

    with jax.named_scope("ph0_max"):
        lax.fori_loop(0, _CPW, p0, 0, unroll=4)
    with jax.named_scope("ph0_pub"):
        for c in range(16):
            pltpu.sync_copy(slab.at[pl.ds(c * _CPW, _CPW)],
                            cm_shared.at[c, pl.ds(s * _CPW, _CPW)])
        plsc.subcore_barrier()

    @pl.when(s < _NC)
    def _():
        with jax.named_scope("ph1_sel"):
            pltpu.sync_copy(cm_shared.at[s], cmbuf)
            v1 = _select64(cmbuf, lambda i: i * 16 + it, _NCHUNK, 11,
                           idchunks)

        def pfetch(q, _):
            rq = idchunks[pl.ds(q, 16)][0]
            pltpu.async_copy(probf_hbm.at[pl.ds(rq * _RW, _RW)],
                             chunkflat.at[pl.ds(q * _RW, _RW)], sem)
            return 0

        with jax.named_scope("ph2_fetch"):
            lax.fori_loop(0, _K, pfetch, 0, unroll=4)
            pltpu.make_async_copy(probf_hbm.at[pl.ds(0, _K * _RW)],
                                  chunkflat.at[pl.ds(0, _K * _RW)],
                                  sem).wait()

        def p2(q, _):
            qv = jnp.zeros((16,), jnp.int32) + q
            v = plsc.load_gather(chunkflat, [q * _RW + it * _NC + s])
            keyb[pl.ds(q * 16, 16)] = plsc.bitcast(_mono_key(v), jnp.int32)
            cid = plsc.load_gather(idchunks, [qv])
            bidxb[pl.ds(q * 16, 16)] = cid * _CH + it
            return 0

        with jax.named_scope("ph2_key"):
            lax.fori_loop(0, _K, p2, 0, unroll=4)
        with jax.named_scope("ph2_sel"):
            _select64(keyb, lambda i: bidxb[pl.ds(i * 16, 16)], _K * 16, 15,
                      zidx, lower=v1)

        with jax.named_scope("ph3_z"):
            ci = pltpu.async_copy(zi_hbm.at[zidx], zrows, sem)
            cj = pltpu.async_copy(zj_hbm.at[zidx], zrows2, sem2)
            ci.wait()
            pltpu.sync_copy(zrows, u_hbm.at[pl.ds(s * 2 * _K, _K)])
            cj.wait()
            pltpu.sync_copy(zrows2, u_hbm.at[pl.ds(s * 2 * _K + _K, _K)])


def _topk_gather_sc(probr, z_i, z_j):
    mesh = plsc.VectorSubcoreMesh(core_axis_name="c", subcore_axis_name="s")
    k = functools.partial(
        pl.kernel,
        mesh=mesh,
        compiler_params=pltpu.CompilerParams(needs_layout_passes=False),
        out_type=jax.ShapeDtypeStruct((_ROWS, _DIM), jnp.float32),
        scratch_types=[
            pltpu.VMEM((_CPW * _RW + 16,), jnp.float32),
            pltpu.VMEM((_NCHUNK,), jnp.int32),
            pltpu.VMEM((_NCHUNK,), jnp.int32),
            pltpu.VMEM((_K * 16,), jnp.int32),
            pltpu.VMEM((_K * 16,), jnp.int32),
            pltpu.VMEM((_K + 16,), jnp.int32),
            pltpu.VMEM((_K,), jnp.int32),
            pltpu.VMEM((_K, _DIM), jnp.float32),
            pltpu.VMEM((_K, _DIM), jnp.float32),
            pltpu.VMEM_SHARED((16, _NCHUNK), jnp.int32),
            pltpu.SemaphoreType.DMA,
            pltpu.SemaphoreType.DMA,
        ],
    )(_sc_body)
    return k(probr, z_i, z_j)


def _loss_body(u_blk_ref, u_all_ref, out_ref):
    i = pl.program_id(0)
    u_blk = u_blk_ref[...]
    u_all = u_all_ref[...]

    n2_all = jnp.sum(u_all * u_all, axis=1, keepdims=True)
    na_all = jnp.sqrt(n2_all)
    n2_blk = jnp.sum(u_blk * u_blk, axis=1, keepdims=True)
    na_blk = jnp.sqrt(n2_blk)

    dots = lax.dot_general(
        u_blk, u_all, (((1,), (1,)), ((), ())),
        preferred_element_type=jnp.float32,
        precision=lax.Precision.HIGHEST,
    )
    denom = jnp.maximum(na_blk * na_all.T, 1e-8)
    e = jnp.exp(dots / denom / _TEMPERATURE)

    gi = lax.broadcasted_iota(jnp.int32, (_BLK, _ROWS), 0) + i * _BLK
    gj = lax.broadcasted_iota(jnp.int32, (_BLK, _ROWS), 1)
    own = (gi // (2 * _K)) == (gj // (2 * _K))
    posm = own & ((gj % (2 * _K)) < _K)

    total = jnp.sum(e, axis=1)
    own_sum = jnp.sum(jnp.where(own, e, 0.0), axis=1)
    pos_sum = jnp.sum(jnp.where(posm, e, 0.0), axis=1)
    part = jnp.sum(jnp.log(total - own_sum) - jnp.log(pos_sum))

    @pl.when(i == 0)
    def _():
        out_ref[0, 0] = 0.0

    out_ref[0, 0] += part


def _loss_tc(u):
    out = pl.pallas_call(
        _loss_body,
        grid=(_ROWS // _BLK,),
        in_specs=[
            pl.BlockSpec((_BLK, _DIM), lambda i: (i, 0)),
            pl.BlockSpec((_ROWS, _DIM), lambda i: (0, 0)),
        ],
        out_specs=pl.BlockSpec(memory_space=pltpu.SMEM),
        out_shape=jax.ShapeDtypeStruct((1, 1), jnp.float32),
    )(u, u)
    return out[0, 0] / _ROWS


def kernel(prob, z_i, z_j):
    probf = prob.reshape(-1)
    u = _topk_gather_sc(probf, z_i, z_j)
    return _loss_tc(u)


# speedup vs baseline: 1.3214x; 1.3214x over previous
import functools

import jax
import jax.numpy as jnp
from jax import lax
from jax.experimental import pallas as pl
from jax.experimental.pallas import tpu as pltpu
from jax.experimental.pallas import tpu_sc as plsc

_TEMPERATURE = 0.5
_K = 64
_NC = 10
_BATCH = 16384
_DIM = 128
_ROWS = 2 * _K * _NC
_BLK = 256
_CH = 16
_NCHUNK = _BATCH // _CH
_CPW = _NCHUNK // 16
_RW = _NC * _CH


def _iota16():
    return lax.iota(jnp.int32, 16)


def _mono_key(v):
    u = lax.bitcast_convert_type(v, jnp.uint32)
    return jnp.where(v < 0.0, ~u, u | jnp.uint32(0x80000000))


def _count_full(nsl, load, pred):
    accs = [jnp.zeros((16,), jnp.int32) for _ in range(4)]
    for i in range(nsl):
        accs[i % 4] = accs[i % 4] + jnp.where(pred(load(i), i), 1, 0)
    return jnp.sum((accs[0] + accs[1]) + (accs[2] + accs[3]))


def _select64(key_ref, idx_vec_fn, n, nbits, out_ref, ckey, cidx):
    nsl = n // 16

    def loadk(i):
        return plsc.bitcast(key_ref[pl.ds(i * 16, 16)], jnp.uint32)

    def body_hi(i, acc):
        bit = lax.shift_left(jnp.uint32(1), (31 - i).astype(jnp.uint32))
        test = acc | bit
        c = _count_full(nsl, loadk, lambda kv, _: kv >= test)
        return jnp.where(c >= 64, test, acc)

    acc = lax.fori_loop(0, 16, body_hi, jnp.uint32(0))
    n_hi = _count_full(nsl, loadk,
                       lambda kv, _: kv > (acc | jnp.uint32(0xFFFF)))

    for i in range(nsl):
        ckey[pl.ds(i * 16, 16)] = jnp.zeros((16,), jnp.int32)
        cidx[pl.ds(i * 16, 16)] = jnp.full((16,), 0x7FFFFFFF, jnp.int32)

    def compact(i, run):
        kv = loadk(i)
        m = (kv & jnp.uint32(0xFFFF0000)) == acc
        incl = plsc.cumsum(m.astype(jnp.int32))
        slots = run + incl - 1
        plsc.store_scatter(ckey, [slots], plsc.bitcast(kv, jnp.int32), mask=m)
        plsc.store_scatter(cidx, [slots], idx_vec_fn(i), mask=m)
        return run + plsc.all_reduce_population_count(m)

    run = lax.fori_loop(0, nsl, compact, jnp.zeros((16,), jnp.int32))
    nsl2 = (jnp.max(run) + 15) >> 4

    def count_c(pred):
        def b(j, cv):
            ck = plsc.bitcast(ckey[pl.ds(j * 16, 16)], jnp.uint32)
            ix = cidx[pl.ds(j * 16, 16)]
            return cv + jnp.where(pred(ck, ix), 1, 0)

        return jnp.sum(lax.fori_loop(0, nsl2, b, jnp.zeros((16,), jnp.int32)))

    def body_lo(i, a2):
        bit = lax.shift_left(jnp.uint32(1), (15 - i).astype(jnp.uint32))
        test = a2 | bit
        c = n_hi + count_c(lambda ck, ix: ck >= test)
        return jnp.where(c >= 64, test, a2)

    v = lax.fori_loop(0, 16, body_lo, acc)
    ngt = n_hi + count_c(lambda ck, ix: ck > v)
    kp = 64 - ngt

    def body_g(i, a2):
        b = (nbits - 1 - i).astype(jnp.int32)
        low = lax.shift_left(jnp.int32(1), b) - 1
        test = a2 | low
        c = count_c(lambda ck, ix: (ck == v) & (ix <= test))
        return jnp.where(c >= kp, a2, a2 | (low + 1))

    g = lax.fori_loop(0, nbits, body_g, jnp.int32(0))

    def emit(i, run2):
        kv = loadk(i)
        idx = idx_vec_fn(i)
        m = (kv > v) | ((kv == v) & (idx <= g))
        incl = plsc.cumsum(m.astype(jnp.int32))
        slots = run2 + incl - 1
        plsc.store_scatter(out_ref, [slots], idx, mask=m)
        return run2 + plsc.all_reduce_population_count(m)

    lax.fori_loop(0, nsl, emit, jnp.zeros((16,), jnp.int32), unroll=2)


def _sc_body(probf_hbm, zi_hbm, zj_hbm, u_hbm,
             chunkflat, slab, cmbuf, keyb, bidxb, idchunks, zidx,
             zrows, zrows2, ckey, cidx, cm_shared, sem, sem2):
    s = lax.axis_index("s")
    it = _iota16()

    with jax.named_scope("ph0_load"):
        pltpu.sync_copy(probf_hbm.at[pl.ds(s * _CPW * _RW, _CPW * _RW)],
                        chunkflat.at[pl.ds(0, _CPW * _RW)])

    def p0(q, _):
        m = chunkflat[pl.ds(q * _RW, 16)]
        for p in range(1, _CH):
            m = jnp.maximum(m, chunkflat[pl.ds(q * _RW + p * _NC, 16)])
        key = plsc.bitcast(_mono_key(m), jnp.int32)
        plsc.store_scatter(slab, [it * _CPW + q], key)
        return 0

    with jax.named_scope("ph0_max"):
        lax.fori_loop(0, _CPW, p0, 0, unroll=4)
    with jax.named_scope("ph0_pub"):
        for c in range(16):
            pltpu.sync_copy(slab.at[pl.ds(c * _CPW, _CPW)],
                            cm_shared.at[c, pl.ds(s * _CPW, _CPW)])
        plsc.subcore_barrier()

    @pl.when(s < _NC)
    def _():
        with jax.named_scope("ph1_sel"):
            pltpu.sync_copy(cm_shared.at[s], cmbuf)
            _select64(cmbuf, lambda i: i * 16 + it, _NCHUNK, 11,
                      idchunks, ckey, cidx)

        def pfetch(q, _):
            rq = idchunks[pl.ds(q, 16)][0]
            pltpu.async_copy(probf_hbm.at[pl.ds(rq * _RW, _RW)],
                             chunkflat.at[pl.ds(q * _RW, _RW)], sem)
            return 0

        with jax.named_scope("ph2_fetch"):
            lax.fori_loop(0, _K, pfetch, 0, unroll=4)
            pltpu.make_async_copy(probf_hbm.at[pl.ds(0, _K * _RW)],
                                  chunkflat.at[pl.ds(0, _K * _RW)],
                                  sem).wait()

        def p2(q, _):
            qv = jnp.zeros((16,), jnp.int32) + q
            v = plsc.load_gather(chunkflat, [q * _RW + it * _NC + s])
            keyb[pl.ds(q * 16, 16)] = plsc.bitcast(_mono_key(v), jnp.int32)
            cid = plsc.load_gather(idchunks, [qv])
            bidxb[pl.ds(q * 16, 16)] = cid * _CH + it
            return 0

        with jax.named_scope("ph2_key"):
            lax.fori_loop(0, _K, p2, 0, unroll=4)
        with jax.named_scope("ph2_sel"):
            _select64(keyb, lambda i: bidxb[pl.ds(i * 16, 16)], _K * 16, 15,
                      zidx, ckey, cidx)

        with jax.named_scope("ph3_z"):
            ci = pltpu.async_copy(zi_hbm.at[zidx], zrows, sem)
            cj = pltpu.async_copy(zj_hbm.at[zidx], zrows2, sem2)
            ci.wait()
            pltpu.sync_copy(zrows, u_hbm.at[pl.ds(s * 2 * _K, _K)])
            cj.wait()
            pltpu.sync_copy(zrows2, u_hbm.at[pl.ds(s * 2 * _K + _K, _K)])


def _topk_gather_sc(probr, z_i, z_j):
    mesh = plsc.VectorSubcoreMesh(core_axis_name="c", subcore_axis_name="s")
    k = functools.partial(
        pl.kernel,
        mesh=mesh,
        compiler_params=pltpu.CompilerParams(needs_layout_passes=False),
        out_type=jax.ShapeDtypeStruct((_ROWS, _DIM), jnp.float32),
        scratch_types=[
            pltpu.VMEM((_CPW * _RW + 16,), jnp.float32),
            pltpu.VMEM((_NCHUNK,), jnp.int32),
            pltpu.VMEM((_NCHUNK,), jnp.int32),
            pltpu.VMEM((_K * 16,), jnp.int32),
            pltpu.VMEM((_K * 16,), jnp.int32),
            pltpu.VMEM((_K + 16,), jnp.int32),
            pltpu.VMEM((_K,), jnp.int32),
            pltpu.VMEM((_K, _DIM), jnp.float32),
            pltpu.VMEM((_K, _DIM), jnp.float32),
            pltpu.VMEM((_NCHUNK,), jnp.int32),
            pltpu.VMEM((_NCHUNK,), jnp.int32),
            pltpu.VMEM_SHARED((16, _NCHUNK), jnp.int32),
            pltpu.SemaphoreType.DMA,
            pltpu.SemaphoreType.DMA,
        ],
    )(_sc_body)
    return k(probr, z_i, z_j)


def _loss_body(u_blk_ref, u_all_ref, out_ref):
    i = pl.program_id(0)
    u_blk = u_blk_ref[...]
    u_all = u_all_ref[...]

    n2_all = jnp.sum(u_all * u_all, axis=1, keepdims=True)
    na_all = jnp.sqrt(n2_all)
    n2_blk = jnp.sum(u_blk * u_blk, axis=1, keepdims=True)
    na_blk = jnp.sqrt(n2_blk)

    dots = lax.dot_general(
        u_blk, u_all, (((1,), (1,)), ((), ())),
        preferred_element_type=jnp.float32,
        precision=lax.Precision.HIGHEST,
    )
    denom = jnp.maximum(na_blk * na_all.T, 1e-8)
    e = jnp.exp(dots / denom / _TEMPERATURE)

    gi = lax.broadcasted_iota(jnp.int32, (_BLK, _ROWS), 0) + i * _BLK
    gj = lax.broadcasted_iota(jnp.int32, (_BLK, _ROWS), 1)
    own = (gi // (2 * _K)) == (gj // (2 * _K))
    posm = own & ((gj % (2 * _K)) < _K)

    total = jnp.sum(e, axis=1)
    own_sum = jnp.sum(jnp.where(own, e, 0.0), axis=1)
    pos_sum = jnp.sum(jnp.where(posm, e, 0.0), axis=1)
    part = jnp.sum(jnp.log(total - own_sum) - jnp.log(pos_sum))

    @pl.when(i == 0)
    def _():
        out_ref[0, 0] = 0.0

    out_ref[0, 0] += part


def _loss_tc(u):
    out = pl.pallas_call(
        _loss_body,
        grid=(_ROWS // _BLK,),
        in_specs=[
            pl.BlockSpec((_BLK, _DIM), lambda i: (i, 0)),
            pl.BlockSpec((_ROWS, _DIM), lambda i: (0, 0)),
        ],
        out_specs=pl.BlockSpec(memory_space=pltpu.SMEM),
        out_shape=jax.ShapeDtypeStruct((1, 1), jnp.float32),
    )(u, u)
    return out[0, 0] / _ROWS


def kernel(prob, z_i, z_j):
    probf = prob.reshape(-1)
    u = _topk_gather_sc(probf, z_i, z_j)
    return _loss_tc(u)


# speedup vs baseline: 1.3477x; 1.0198x over previous
import functools

import jax
import jax.numpy as jnp
from jax import lax
from jax.experimental import pallas as pl
from jax.experimental.pallas import tpu as pltpu
from jax.experimental.pallas import tpu_sc as plsc

_TEMPERATURE = 0.5
_K = 64
_NC = 10
_BATCH = 16384
_DIM = 128
_ROWS = 2 * _K * _NC
_BLK = 640
_CH = 16
_NCHUNK = _BATCH // _CH
_CPW = _NCHUNK // 16
_RW = _NC * _CH


def _iota16():
    return lax.iota(jnp.int32, 16)


def _mono_key(v):
    u = lax.bitcast_convert_type(v, jnp.uint32)
    return jnp.where(v < 0.0, ~u, u | jnp.uint32(0x80000000))


def _count_full(nsl, load, pred):
    accs = [jnp.zeros((16,), jnp.int32) for _ in range(4)]
    for i in range(nsl):
        accs[i % 4] = accs[i % 4] + jnp.where(pred(load(i), i), 1, 0)
    return jnp.sum((accs[0] + accs[1]) + (accs[2] + accs[3]))


def _select64(key_ref, idx_vec_fn, n, nbits, out_ref, ckey, cidx):
    nsl = n // 16

    def loadk(i):
        return plsc.bitcast(key_ref[pl.ds(i * 16, 16)], jnp.uint32)

    def body_hi(i, acc):
        bit = lax.shift_left(jnp.uint32(1), (31 - i).astype(jnp.uint32))
        test = acc | bit
        c = _count_full(nsl, loadk, lambda kv, _: kv >= test)
        return jnp.where(c >= 64, test, acc)

    acc = lax.fori_loop(0, 16, body_hi, jnp.uint32(0))
    n_hi = _count_full(nsl, loadk,
                       lambda kv, _: kv > (acc | jnp.uint32(0xFFFF)))

    for i in range(nsl):
        ckey[pl.ds(i * 16, 16)] = jnp.zeros((16,), jnp.int32)
        cidx[pl.ds(i * 16, 16)] = jnp.full((16,), 0x7FFFFFFF, jnp.int32)

    def compact(i, run):
        kv = loadk(i)
        m = (kv & jnp.uint32(0xFFFF0000)) == acc
        incl = plsc.cumsum(m.astype(jnp.int32))
        slots = run + incl - 1
        plsc.store_scatter(ckey, [slots], plsc.bitcast(kv, jnp.int32), mask=m)
        plsc.store_scatter(cidx, [slots], idx_vec_fn(i), mask=m)
        return run + plsc.all_reduce_population_count(m)

    run = lax.fori_loop(0, nsl, compact, jnp.zeros((16,), jnp.int32))
    nsl2 = (jnp.max(run) + 15) >> 4

    def count_c(pred):
        def b(j, cv):
            ck = plsc.bitcast(ckey[pl.ds(j * 16, 16)], jnp.uint32)
            ix = cidx[pl.ds(j * 16, 16)]
            return cv + jnp.where(pred(ck, ix), 1, 0)

        return jnp.sum(lax.fori_loop(0, nsl2, b, jnp.zeros((16,), jnp.int32)))

    def body_lo(i, a2):
        bit = lax.shift_left(jnp.uint32(1), (15 - i).astype(jnp.uint32))
        test = a2 | bit
        c = n_hi + count_c(lambda ck, ix: ck >= test)
        return jnp.where(c >= 64, test, a2)

    v = lax.fori_loop(0, 16, body_lo, acc)
    ngt = n_hi + count_c(lambda ck, ix: ck > v)
    kp = 64 - ngt

    def body_g(i, a2):
        b = (nbits - 1 - i).astype(jnp.int32)
        low = lax.shift_left(jnp.int32(1), b) - 1
        test = a2 | low
        c = count_c(lambda ck, ix: (ck == v) & (ix <= test))
        return jnp.where(c >= kp, a2, a2 | (low + 1))

    g = lax.fori_loop(0, nbits, body_g, jnp.int32(0))

    def emit(i, run2):
        kv = loadk(i)
        idx = idx_vec_fn(i)
        m = (kv > v) | ((kv == v) & (idx <= g))
        incl = plsc.cumsum(m.astype(jnp.int32))
        slots = run2 + incl - 1
        plsc.store_scatter(out_ref, [slots], idx, mask=m)
        return run2 + plsc.all_reduce_population_count(m)

    lax.fori_loop(0, nsl, emit, jnp.zeros((16,), jnp.int32), unroll=2)


def _sc_body(probf_hbm, zi_hbm, zj_hbm, u_hbm,
             chunkflat, slab, cmbuf, keyb, bidxb, idchunks, zidx,
             zrows, zrows2, ckey, cidx, cm_shared, sem, sem2):
    s = lax.axis_index("s")
    it = _iota16()

    with jax.named_scope("ph0_load"):
        pltpu.sync_copy(probf_hbm.at[pl.ds(s * _CPW * _RW, _CPW * _RW)],
                        chunkflat.at[pl.ds(0, _CPW * _RW)])

    def p0(q, _):
        m = chunkflat[pl.ds(q * _RW, 16)]
        for p in range(1, _CH):
            m = jnp.maximum(m, chunkflat[pl.ds(q * _RW + p * _NC, 16)])
        key = plsc.bitcast(_mono_key(m), jnp.int32)
        plsc.store_scatter(slab, [it * _CPW + q], key)
        return 0

    with jax.named_scope("ph0_max"):
        lax.fori_loop(0, _CPW, p0, 0, unroll=4)
    with jax.named_scope("ph0_pub"):
        for c in range(16):
            pltpu.sync_copy(slab.at[pl.ds(c * _CPW, _CPW)],
                            cm_shared.at[c, pl.ds(s * _CPW, _CPW)])
        plsc.subcore_barrier()

    @pl.when(s < _NC)
    def _():
        with jax.named_scope("ph1_sel"):
            pltpu.sync_copy(cm_shared.at[s], cmbuf)
            _select64(cmbuf, lambda i: i * 16 + it, _NCHUNK, 11,
                      idchunks, ckey, cidx)

        def pfetch(q, _):
            rq = idchunks[pl.ds(q, 16)][0]
            pltpu.async_copy(probf_hbm.at[pl.ds(rq * _RW, _RW)],
                             chunkflat.at[pl.ds(q * _RW, _RW)], sem)
            return 0

        with jax.named_scope("ph2_fetch"):
            lax.fori_loop(0, _K, pfetch, 0, unroll=4)
            pltpu.make_async_copy(probf_hbm.at[pl.ds(0, _K * _RW)],
                                  chunkflat.at[pl.ds(0, _K * _RW)],
                                  sem).wait()

        def p2(q, _):
            qv = jnp.zeros((16,), jnp.int32) + q
            v = plsc.load_gather(chunkflat, [q * _RW + it * _NC + s])
            keyb[pl.ds(q * 16, 16)] = plsc.bitcast(_mono_key(v), jnp.int32)
            cid = plsc.load_gather(idchunks, [qv])
            bidxb[pl.ds(q * 16, 16)] = cid * _CH + it
            return 0

        with jax.named_scope("ph2_key"):
            lax.fori_loop(0, _K, p2, 0, unroll=4)
        with jax.named_scope("ph2_sel"):
            _select64(keyb, lambda i: bidxb[pl.ds(i * 16, 16)], _K * 16, 15,
                      zidx, ckey, cidx)

        with jax.named_scope("ph3_z"):
            ci = pltpu.async_copy(zi_hbm.at[zidx], zrows, sem)
            cj = pltpu.async_copy(zj_hbm.at[zidx], zrows2, sem2)
            ci.wait()
            pltpu.sync_copy(zrows, u_hbm.at[pl.ds(s * 2 * _K, _K)])
            cj.wait()
            pltpu.sync_copy(zrows2, u_hbm.at[pl.ds(s * 2 * _K + _K, _K)])


def _topk_gather_sc(probr, z_i, z_j):
    mesh = plsc.VectorSubcoreMesh(core_axis_name="c", subcore_axis_name="s")
    k = functools.partial(
        pl.kernel,
        mesh=mesh,
        compiler_params=pltpu.CompilerParams(needs_layout_passes=False),
        out_type=jax.ShapeDtypeStruct((_ROWS, _DIM), jnp.float32),
        scratch_types=[
            pltpu.VMEM((_CPW * _RW + 16,), jnp.float32),
            pltpu.VMEM((_NCHUNK,), jnp.int32),
            pltpu.VMEM((_NCHUNK,), jnp.int32),
            pltpu.VMEM((_K * 16,), jnp.int32),
            pltpu.VMEM((_K * 16,), jnp.int32),
            pltpu.VMEM((_K + 16,), jnp.int32),
            pltpu.VMEM((_K,), jnp.int32),
            pltpu.VMEM((_K, _DIM), jnp.float32),
            pltpu.VMEM((_K, _DIM), jnp.float32),
            pltpu.VMEM((_NCHUNK,), jnp.int32),
            pltpu.VMEM((_NCHUNK,), jnp.int32),
            pltpu.VMEM_SHARED((16, _NCHUNK), jnp.int32),
            pltpu.SemaphoreType.DMA,
            pltpu.SemaphoreType.DMA,
        ],
    )(_sc_body)
    return k(probr, z_i, z_j)


def _loss_body(u_blk_ref, u_all_ref, out_ref):
    i = pl.program_id(0)
    u_blk = u_blk_ref[...]
    u_all = u_all_ref[...]

    n2_all = jnp.sum(u_all * u_all, axis=1, keepdims=True)
    na_all = jnp.sqrt(n2_all)
    n2_blk = jnp.sum(u_blk * u_blk, axis=1, keepdims=True)
    na_blk = jnp.sqrt(n2_blk)

    dots = lax.dot_general(
        u_blk, u_all, (((1,), (1,)), ((), ())),
        preferred_element_type=jnp.float32,
        precision=lax.Precision.HIGHEST,
    )
    denom = jnp.maximum(na_blk * na_all.T, 1e-8)
    e = jnp.exp(dots / denom / _TEMPERATURE)

    gi = lax.broadcasted_iota(jnp.int32, (_BLK, _ROWS), 0) + i * _BLK
    gj = lax.broadcasted_iota(jnp.int32, (_BLK, _ROWS), 1)
    own = (gi // (2 * _K)) == (gj // (2 * _K))
    posm = own & ((gj % (2 * _K)) < _K)

    total = jnp.sum(e, axis=1)
    own_sum = jnp.sum(jnp.where(own, e, 0.0), axis=1)
    pos_sum = jnp.sum(jnp.where(posm, e, 0.0), axis=1)
    part = jnp.sum(jnp.log(total - own_sum) - jnp.log(pos_sum))

    @pl.when(i == 0)
    def _():
        out_ref[0, 0] = 0.0

    out_ref[0, 0] += part


def _loss_tc(u):
    out = pl.pallas_call(
        _loss_body,
        grid=(_ROWS // _BLK,),
        in_specs=[
            pl.BlockSpec((_BLK, _DIM), lambda i: (i, 0)),
            pl.BlockSpec((_ROWS, _DIM), lambda i: (0, 0)),
        ],
        out_specs=pl.BlockSpec(memory_space=pltpu.SMEM),
        out_shape=jax.ShapeDtypeStruct((1, 1), jnp.float32),
    )(u, u)
    return out[0, 0] / _ROWS


def kernel(prob, z_i, z_j):
    probf = prob.reshape(-1)
    u = _topk_gather_sc(probf, z_i, z_j)
    return _loss_tc(u)
